# Initial kernel scaffold; baseline (speedup 1.0000x reference)
import functools

import jax
import jax.numpy as jnp
from jax import lax
from jax.experimental import pallas as pl
from jax.experimental.pallas import tpu as pltpu
from jax.experimental.pallas import tpu_sc as plsc

N = 10000
E = 320000
DF = 128
NC, NS = 2, 16
NW = NC * NS
CH = 128
NCHUNK = E // CH
RPT = N // NS

_mesh = plsc.VectorSubcoreMesh(
    core_axis_name="c", subcore_axis_name="s", num_cores=NC, num_subcores=NS
)


@functools.partial(
    pl.kernel,
    out_type=jax.ShapeDtypeStruct((NC, N, 16), jnp.float32),
    mesh=_mesh,
    scratch_types=[
        pltpu.VMEM_SHARED((N, 16), jnp.float32),
        pltpu.VMEM((RPT, 16), jnp.float32),
        pltpu.VMEM((CH, 16), jnp.float32),
        pltpu.VMEM((1, CH), jnp.int32),
    ],
)
def _hist_kernel(dst_hbm, out_hbm, hist, zbuf, ones_v, didx):
    c = lax.axis_index("c")
    s = lax.axis_index("s")
    wid = c * NS + s

    @pl.loop(0, RPT)
    def _(i):
        zbuf[i, :] = jnp.zeros((16,), jnp.float32)

    @pl.loop(0, CH)
    def _(i):
        ones_v[i, :] = jnp.full((16,), 1.0, jnp.float32)

    pltpu.sync_copy(zbuf, hist.at[pl.ds(s * RPT, RPT)])
    plsc.subcore_barrier()

    @pl.loop(wid, NCHUNK, step=NW)
    def _(j):
        pltpu.sync_copy(dst_hbm.at[j], didx.at[0])
        pltpu.sync_copy(ones_v, hist.at[didx.at[0]], add=True)

    plsc.subcore_barrier()
    pltpu.sync_copy(
        hist.at[pl.ds(s * RPT, RPT)], out_hbm.at[c, pl.ds(s * RPT, RPT)]
    )


@functools.partial(
    pl.kernel,
    out_type=jax.ShapeDtypeStruct((NC, N, DF), jnp.float32),
    mesh=_mesh,
    scratch_types=[
        pltpu.VMEM_SHARED((N, DF), jnp.float32),
        pltpu.VMEM((1, CH), jnp.int32),
        pltpu.VMEM((1, CH), jnp.int32),
        pltpu.VMEM((CH, DF), jnp.float32),
        pltpu.SemaphoreType.DMA,
    ],
)
def _agg_kernel(src_hbm, dst_hbm, g_hbm, out_hbm, acc, sidx, didx, rows, sem):
    c = lax.axis_index("c")
    s = lax.axis_index("s")
    wid = c * NS + s
    r0 = s * RPT

    pltpu.sync_copy(g_hbm.at[pl.ds(r0, RPT)], acc.at[pl.ds(r0, RPT)])
    plsc.subcore_barrier()

    @pl.loop(wid, NCHUNK, step=NW)
    def _(j):
        pltpu.sync_copy(src_hbm.at[j], sidx.at[0])
        pltpu.sync_copy(dst_hbm.at[j], didx.at[0])
        pltpu.async_copy(g_hbm.at[sidx.at[0]], rows, sem).wait()
        pltpu.sync_copy(rows, acc.at[didx.at[0]], add=True)

    plsc.subcore_barrier()
    pltpu.sync_copy(acc.at[pl.ds(r0, RPT)], out_hbm.at[c, pl.ds(r0, RPT)])


_BR = 1000


def _mm_body(x_ref, w_ref, hist_ref, g_ref):
    h = jnp.dot(x_ref[...], w_ref[...], preferred_element_type=jnp.float32)
    deg = hist_ref[0, :, 0:1] + hist_ref[1, :, 0:1] + 1.0
    g_ref[...] = h * lax.rsqrt(deg)


def _fin_body(p_ref, g_ref, hist_ref, b_ref, o_ref):
    deg = hist_ref[0, :, 0:1] + hist_ref[1, :, 0:1] + 1.0
    dinv = lax.rsqrt(deg)
    agg = p_ref[0] + p_ref[1] - g_ref[...]
    o_ref[...] = jnp.maximum(agg * dinv + b_ref[...], 0.0)


def kernel(x, edge_index, W, b):
    src = edge_index[0].astype(jnp.int32).reshape(NCHUNK, CH)
    dst = edge_index[1].astype(jnp.int32).reshape(NCHUNK, CH)

    hist = _hist_kernel(dst)

    g = pl.pallas_call(
        _mm_body,
        grid=(N // _BR,),
        in_specs=[
            pl.BlockSpec((_BR, DF), lambda i: (i, 0)),
            pl.BlockSpec((DF, DF), lambda i: (0, 0)),
            pl.BlockSpec((NC, _BR, 16), lambda i: (0, i, 0)),
        ],
        out_specs=pl.BlockSpec((_BR, DF), lambda i: (i, 0)),
        out_shape=jax.ShapeDtypeStruct((N, DF), jnp.float32),
    )(x, W, hist)

    p = _agg_kernel(src, dst, g)

    out = pl.pallas_call(
        _fin_body,
        grid=(N // _BR,),
        in_specs=[
            pl.BlockSpec((NC, _BR, DF), lambda i: (0, i, 0)),
            pl.BlockSpec((_BR, DF), lambda i: (i, 0)),
            pl.BlockSpec((NC, _BR, 16), lambda i: (0, i, 0)),
            pl.BlockSpec((1, DF), lambda i: (0, 0)),
        ],
        out_specs=pl.BlockSpec((_BR, DF), lambda i: (i, 0)),
        out_shape=jax.ShapeDtypeStruct((N, DF), jnp.float32),
    )(p, g, hist, b.reshape(1, DF))

    return out


# speedup vs baseline: 21.3283x; 21.3283x over previous
import functools

import jax
import jax.numpy as jnp
from jax import lax
from jax.experimental import pallas as pl
from jax.experimental.pallas import tpu as pltpu
from jax.experimental.pallas import tpu_sc as plsc

N = 10000
E = 320000
DF = 128
NC, NS = 2, 16
NW = NC * NS
CH = 128
NCHUNK = E // CH
RPT = 624
TAIL = N - NS * RPT

_mesh = plsc.VectorSubcoreMesh(
    core_axis_name="c", subcore_axis_name="s", num_cores=NC, num_subcores=NS
)


@functools.partial(
    pl.kernel,
    out_type=jax.ShapeDtypeStruct((NC, N, 16), jnp.float32),
    mesh=_mesh,
    scratch_types=[
        pltpu.VMEM_SHARED((N, 16), jnp.float32),
        pltpu.VMEM((CH, 16), jnp.float32),
        pltpu.VMEM((1, CH), jnp.int32),
    ],
    compiler_params=pltpu.CompilerParams(use_tc_tiling_on_sc=False),
)
def _hist_kernel(dst_hbm, ones_hbm, zeros_hbm, out_hbm, hist, ones_v, didx):
    c = lax.axis_index("c")
    s = lax.axis_index("s")
    wid = c * NS + s

    pltpu.sync_copy(ones_hbm, ones_v)
    pltpu.sync_copy(
        zeros_hbm.at[pl.ds(s * RPT, RPT)], hist.at[pl.ds(s * RPT, RPT)]
    )

    @pl.when(s == NS - 1)
    def _():
        pltpu.sync_copy(
            zeros_hbm.at[pl.ds(NS * RPT, TAIL)], hist.at[pl.ds(NS * RPT, TAIL)]
        )

    plsc.subcore_barrier()

    @pl.loop(wid, NCHUNK, step=NW)
    def _(j):
        pltpu.sync_copy(dst_hbm.at[j], didx.at[0])
        pltpu.sync_copy(ones_v, hist.at[didx.at[0]], add=True)

    plsc.subcore_barrier()
    pltpu.sync_copy(
        hist.at[pl.ds(s * RPT, RPT)], out_hbm.at[c, pl.ds(s * RPT, RPT)]
    )

    @pl.when(s == NS - 1)
    def _():
        pltpu.sync_copy(
            hist.at[pl.ds(NS * RPT, TAIL)], out_hbm.at[c, pl.ds(NS * RPT, TAIL)]
        )


@functools.partial(
    pl.kernel,
    out_type=jax.ShapeDtypeStruct((NC, N, DF), jnp.float32),
    mesh=_mesh,
    scratch_types=[
        pltpu.VMEM_SHARED((N, DF), jnp.float32),
        pltpu.VMEM((1, CH), jnp.int32),
        pltpu.VMEM((1, CH), jnp.int32),
        pltpu.VMEM((CH, DF), jnp.float32),
        pltpu.SemaphoreType.DMA,
    ],
)
def _agg_kernel(src_hbm, dst_hbm, g_hbm, out_hbm, acc, sidx, didx, rows, sem):
    c = lax.axis_index("c")
    s = lax.axis_index("s")
    wid = c * NS + s
    r0 = s * RPT

    pltpu.sync_copy(g_hbm.at[pl.ds(r0, RPT)], acc.at[pl.ds(r0, RPT)])

    @pl.when(s == NS - 1)
    def _():
        pltpu.sync_copy(g_hbm.at[pl.ds(NS * RPT, TAIL)], acc.at[pl.ds(NS * RPT, TAIL)])

    plsc.subcore_barrier()

    @pl.loop(wid, NCHUNK, step=NW)
    def _(j):
        pltpu.sync_copy(src_hbm.at[j], sidx.at[0])
        pltpu.sync_copy(dst_hbm.at[j], didx.at[0])
        pltpu.async_copy(g_hbm.at[sidx.at[0]], rows, sem).wait()
        pltpu.sync_copy(rows, acc.at[didx.at[0]], add=True)

    plsc.subcore_barrier()
    pltpu.sync_copy(acc.at[pl.ds(r0, RPT)], out_hbm.at[c, pl.ds(r0, RPT)])

    @pl.when(s == NS - 1)
    def _():
        pltpu.sync_copy(
            acc.at[pl.ds(NS * RPT, TAIL)], out_hbm.at[c, pl.ds(NS * RPT, TAIL)]
        )


_BR = 1000


def _mm_body(x_ref, w_ref, hist_ref, g_ref):
    h = jnp.dot(x_ref[...], w_ref[...], preferred_element_type=jnp.float32)
    deg = hist_ref[0, :, 0:1] + hist_ref[1, :, 0:1] + 1.0
    g_ref[...] = h * lax.rsqrt(deg)


def _fin_body(p_ref, g_ref, hist_ref, b_ref, o_ref):
    deg = hist_ref[0, :, 0:1] + hist_ref[1, :, 0:1] + 1.0
    dinv = lax.rsqrt(deg)
    agg = p_ref[0] + p_ref[1] - g_ref[...]
    o_ref[...] = jnp.maximum(agg * dinv + b_ref[...], 0.0)


def kernel(x, edge_index, W, b):
    src = edge_index[0].astype(jnp.int32).reshape(NCHUNK, CH)
    dst = edge_index[1].astype(jnp.int32).reshape(NCHUNK, CH)

    ones16 = jnp.ones((CH, 16), jnp.float32)
    zeros16 = jnp.zeros((N, 16), jnp.float32)
    hist = _hist_kernel(dst, ones16, zeros16)

    g = pl.pallas_call(
        _mm_body,
        grid=(N // _BR,),
        in_specs=[
            pl.BlockSpec((_BR, DF), lambda i: (i, 0)),
            pl.BlockSpec((DF, DF), lambda i: (0, 0)),
            pl.BlockSpec((NC, _BR, 16), lambda i: (0, i, 0)),
        ],
        out_specs=pl.BlockSpec((_BR, DF), lambda i: (i, 0)),
        out_shape=jax.ShapeDtypeStruct((N, DF), jnp.float32),
    )(x, W, hist)

    p = _agg_kernel(src, dst, g)

    out = pl.pallas_call(
        _fin_body,
        grid=(N // _BR,),
        in_specs=[
            pl.BlockSpec((NC, _BR, DF), lambda i: (0, i, 0)),
            pl.BlockSpec((_BR, DF), lambda i: (i, 0)),
            pl.BlockSpec((NC, _BR, 16), lambda i: (0, i, 0)),
            pl.BlockSpec((1, DF), lambda i: (0, 0)),
        ],
        out_specs=pl.BlockSpec((_BR, DF), lambda i: (i, 0)),
        out_shape=jax.ShapeDtypeStruct((N, DF), jnp.float32),
    )(p, g, hist, b.reshape(1, DF))

    return out


# speedup vs baseline: 26.0085x; 1.2194x over previous
import functools

import jax
import jax.numpy as jnp
import numpy as np
from jax import lax
from jax.experimental import pallas as pl
from jax.experimental.pallas import tpu as pltpu
from jax.experimental.pallas import tpu_sc as plsc

N = 10000
E = 320000
DF = 128
NC, NS = 2, 16
NW = NC * NS
CH = 112
CPT = 90
EPAD = NW * CPT * CH - E
NACC = 10008
RPT = 624
TAIL = N - NS * RPT
ATAIL = NACC - NS * RPT
NBUF = 2
HCPT = 79
HPAD = NW * HCPT * 128 - E

_mesh = plsc.VectorSubcoreMesh(
    core_axis_name="c", subcore_axis_name="s", num_cores=NC, num_subcores=NS
)

_sc_params = pltpu.CompilerParams(use_tc_tiling_on_sc=False)


@functools.partial(
    pl.kernel,
    out_type=jax.ShapeDtypeStruct((NC, N, 16), jnp.float32),
    mesh=_mesh,
    scratch_types=[
        pltpu.VMEM_SHARED((NACC, 16), jnp.float32),
        pltpu.VMEM((128, 16), jnp.float32),
        pltpu.VMEM((HCPT, 128), jnp.int32),
        pltpu.SemaphoreType.DMA,
    ],
    compiler_params=_sc_params,
)
def _hist_kernel(dst_hbm, ones_hbm, zeros_hbm, out_hbm, hist, ones_v, didx, sem):
    c = lax.axis_index("c")
    s = lax.axis_index("s")
    wid = c * NS + s

    pltpu.sync_copy(ones_hbm, ones_v)
    pltpu.sync_copy(dst_hbm.at[wid], didx)
    pltpu.sync_copy(
        zeros_hbm.at[pl.ds(s * RPT, RPT)], hist.at[pl.ds(s * RPT, RPT)]
    )

    @pl.when(s == NS - 1)
    def _():
        pltpu.sync_copy(
            zeros_hbm.at[pl.ds(NS * RPT, TAIL)], hist.at[pl.ds(NS * RPT, TAIL)]
        )

    plsc.subcore_barrier()

    @pl.loop(0, HCPT)
    def _(j):
        pltpu.async_copy(ones_v, hist.at[didx.at[j]], sem, add=True)

    @pl.loop(0, HCPT)
    def _(j):
        pltpu.make_async_copy(ones_v, hist.at[didx.at[j]], sem).wait()

    plsc.subcore_barrier()
    pltpu.sync_copy(
        hist.at[pl.ds(s * RPT, RPT)], out_hbm.at[c, pl.ds(s * RPT, RPT)]
    )

    @pl.when(s == NS - 1)
    def _():
        pltpu.sync_copy(
            hist.at[pl.ds(NS * RPT, TAIL)], out_hbm.at[c, pl.ds(NS * RPT, TAIL)]
        )


@functools.partial(
    pl.kernel,
    out_type=jax.ShapeDtypeStruct((NC, NACC, DF), jnp.float32),
    mesh=_mesh,
    scratch_types=[
        pltpu.VMEM_SHARED((NACC, DF), jnp.float32),
        pltpu.VMEM((CPT, CH), jnp.int32),
        pltpu.VMEM((CPT, CH), jnp.int32),
        [pltpu.VMEM((CH, DF), jnp.float32)] * NBUF,
        [pltpu.SemaphoreType.DMA] * NBUF,
    ],
    compiler_params=_sc_params,
)
def _agg_kernel(src_hbm, dst_hbm, g_hbm, out_hbm, acc, sidx, didx, rows, sems):
    c = lax.axis_index("c")
    s = lax.axis_index("s")
    wid = c * NS + s
    r0 = s * RPT

    pltpu.sync_copy(src_hbm.at[wid], sidx)
    pltpu.sync_copy(dst_hbm.at[wid], didx)

    def hbm_to_spmem(src, dst_sl, nrows):
        pltpu.sync_copy(src, rows[0].at[pl.ds(0, nrows)])
        pltpu.sync_copy(rows[0].at[pl.ds(0, nrows)], acc.at[dst_sl])

    for q in range(RPT // 104):
        sl = pl.ds(r0 + q * 104, 104)
        hbm_to_spmem(g_hbm.at[sl], sl, 104)

    @pl.when(s == NS - 1)
    def _():
        hbm_to_spmem(
            g_hbm.at[pl.ds(NS * RPT, TAIL)], pl.ds(NS * RPT, TAIL), TAIL
        )

    plsc.subcore_barrier()

    def start_gather(j, b):
        pltpu.async_copy(g_hbm.at[sidx.at[j]], rows[b], sems[b])

    def wait_gather(j, b):
        pltpu.make_async_copy(g_hbm.at[sidx.at[j]], rows[b], sems[b]).wait()

    for b in range(NBUF):
        start_gather(b, b)

    @pl.loop(0, CPT, step=NBUF)
    def _(j0):
        for u in range(NBUF):
            j = j0 + u
            wait_gather(j, u)
            pltpu.sync_copy(rows[u], acc.at[didx.at[j]], add=True)
            nxt = j + NBUF

            @pl.when(nxt < CPT)
            def _():
                start_gather(nxt, u)

    plsc.subcore_barrier()

    def spmem_to_hbm(src_sl, nrows):
        pltpu.sync_copy(acc.at[src_sl], rows[0].at[pl.ds(0, nrows)])
        pltpu.sync_copy(rows[0].at[pl.ds(0, nrows)], out_hbm.at[c, src_sl])

    for q in range(RPT // 104):
        spmem_to_hbm(pl.ds(r0 + q * 104, 104), 104)

    @pl.when(s == NS - 1)
    def _():
        spmem_to_hbm(pl.ds(NS * RPT, ATAIL), ATAIL)


_BR = 1000


def _mm_body(x_ref, w_ref, hist_ref, g_ref):
    h = jnp.dot(x_ref[...], w_ref[...], preferred_element_type=jnp.float32)
    deg = hist_ref[0, :, 0:1] + hist_ref[1, :, 0:1] + 1.0
    g_ref[...] = h * lax.rsqrt(deg)


def _fin_body(p_ref, g_ref, hist_ref, b_ref, o_ref):
    deg = hist_ref[0, :, 0:1] + hist_ref[1, :, 0:1] + 1.0
    dinv = lax.rsqrt(deg)
    agg = p_ref[0] + p_ref[1] - g_ref[...]
    o_ref[...] = jnp.maximum(agg * dinv + b_ref[...], 0.0)


_SRC_PAD = np.zeros((EPAD,), np.int32)
_DST_PAD = np.full((EPAD,), N, np.int32)
_HDST_PAD = np.full((HPAD,), N, np.int32)


def kernel(x, edge_index, W, b):
    src_flat = edge_index[0].astype(jnp.int32)
    dst_flat = edge_index[1].astype(jnp.int32)
    src = jnp.concatenate([src_flat, _SRC_PAD]).reshape(NW, CPT, CH)
    dst = jnp.concatenate([dst_flat, _DST_PAD]).reshape(NW, CPT, CH)
    hdst = jnp.concatenate([dst_flat, _HDST_PAD]).reshape(NW, HCPT, 128)

    ones16 = jnp.ones((128, 16), jnp.float32)
    zeros16 = jnp.zeros((N, 16), jnp.float32)
    hist = _hist_kernel(hdst, ones16, zeros16)

    g = pl.pallas_call(
        _mm_body,
        grid=(N // _BR,),
        in_specs=[
            pl.BlockSpec((_BR, DF), lambda i: (i, 0)),
            pl.BlockSpec((DF, DF), lambda i: (0, 0)),
            pl.BlockSpec((NC, _BR, 16), lambda i: (0, i, 0)),
        ],
        out_specs=pl.BlockSpec((_BR, DF), lambda i: (i, 0)),
        out_shape=jax.ShapeDtypeStruct((N, DF), jnp.float32),
    )(x, W, hist)

    p = _agg_kernel(src, dst, g)

    out = pl.pallas_call(
        _fin_body,
        grid=(N // _BR,),
        in_specs=[
            pl.BlockSpec((NC, _BR, DF), lambda i: (0, i, 0)),
            pl.BlockSpec((_BR, DF), lambda i: (i, 0)),
            pl.BlockSpec((NC, _BR, 16), lambda i: (0, i, 0)),
            pl.BlockSpec((1, DF), lambda i: (0, 0)),
        ],
        out_specs=pl.BlockSpec((_BR, DF), lambda i: (i, 0)),
        out_shape=jax.ShapeDtypeStruct((N, DF), jnp.float32),
    )(p, g, hist, b.reshape(1, DF))

    return out
